# baseline (speedup 1.0000x reference)
import functools
import math

import jax
import jax.numpy as jnp
from jax.experimental import pallas as pl
from jax.experimental.pallas import tpu as pltpu

B = 2
S = 2048
D_MODEL = 768
D_FF = 2048
E = 64
CAP_FACTOR = 1.25
C = int(math.ceil(B * S * CAP_FACTOR / E))

S_CHUNKS = 8
S_BLK = S // S_CHUNKS
F_CHUNKS = 8
F_BLK = D_FF // F_CHUNKS


def _router_ln_kernel(x_ref, rw_ref, rb_ref, g_ref, bb_ref, e_ref, o_ref,
                      acc_ref):
    s = pl.program_id(1)
    xx = x_ref[0]
    part = jnp.sum(xx, axis=0, keepdims=True)

    @pl.when(s == 0)
    def _():
        acc_ref[...] = part

    @pl.when(s != 0)
    def _():
        acc_ref[...] += part

    mu = jnp.mean(xx, axis=1, keepdims=True)
    var = jnp.mean((xx - mu) ** 2, axis=1, keepdims=True)
    o_ref[0] = (xx - mu) * jax.lax.rsqrt(var + 1e-5) * g_ref[...] + bb_ref[...]

    @pl.when(s == S_CHUNKS - 1)
    def _():
        pooled = acc_ref[...] * (1.0 / S)
        logits = jnp.dot(pooled, rw_ref[...],
                         preferred_element_type=jnp.float32) + rb_ref[...]
        maxv = jnp.max(logits, axis=1, keepdims=True)
        idx = jax.lax.broadcasted_iota(jnp.int32, (1, E), 1)
        masked = jnp.where(logits >= maxv, idx, jnp.int32(E))
        am = jnp.min(masked, axis=1, keepdims=True)
        e_ref[0] = jnp.broadcast_to(am, (1, 128))


def _expert_kernel(e_ref, x_ref, w1_ref, b1_ref, w2_ref, b2_ref, g_ref,
                   bb_ref, prev_ref, o_ref, acc_ref):
    del prev_ref
    b = pl.program_id(0)
    f = pl.program_id(1)
    xc = x_ref[0]
    h = jnp.maximum(
        jnp.dot(xc, w1_ref[0], preferred_element_type=jnp.float32)
        + b1_ref[0], 0.0)
    part = jnp.dot(h, w2_ref[0], preferred_element_type=jnp.float32)

    @pl.when(f == 0)
    def _():
        acc_ref[...] = part

    @pl.when(f != 0)
    def _():
        acc_ref[...] += part

    @pl.when(f == F_CHUNKS - 1)
    def _():
        y = acc_ref[...] + b2_ref[0]
        valid = jnp.logical_or(b == 0, e_ref[0, 0, 0] != e_ref[1, 0, 0])
        res = xc + jnp.where(valid, y, 0.0)
        mu = jnp.mean(res, axis=1, keepdims=True)
        var = jnp.mean((res - mu) ** 2, axis=1, keepdims=True)
        o_ref[0] = ((res - mu) * jax.lax.rsqrt(var + 1e-5) * g_ref[...]
                    + bb_ref[...])


@functools.partial(jax.jit, static_argnames=("interpret",))
def _run(x, router_w, router_b, W1, b1, W2, b2, ln_g, ln_b, interpret=False):
    rb2 = router_b.reshape(1, E)
    g2 = ln_g.reshape(1, D_MODEL)
    lb2 = ln_b.reshape(1, D_MODEL)
    b1r = b1.reshape(E, 1, D_FF)
    b2r = b2.reshape(E, 1, D_MODEL)

    e_out, out_a = pl.pallas_call(
        _router_ln_kernel,
        grid=(B, S_CHUNKS),
        in_specs=[
            pl.BlockSpec((1, S_BLK, D_MODEL), lambda b, s: (b, s, 0)),
            pl.BlockSpec((D_MODEL, E), lambda b, s: (0, 0)),
            pl.BlockSpec((1, E), lambda b, s: (0, 0)),
            pl.BlockSpec((1, D_MODEL), lambda b, s: (0, 0)),
            pl.BlockSpec((1, D_MODEL), lambda b, s: (0, 0)),
        ],
        out_specs=[
            pl.BlockSpec((1, 1, 128), lambda b, s: (b, 0, 0)),
            pl.BlockSpec((1, S_BLK, D_MODEL), lambda b, s: (b, s, 0)),
        ],
        out_shape=[
            jax.ShapeDtypeStruct((B, 1, 128), jnp.int32),
            jax.ShapeDtypeStruct((B, S, D_MODEL), jnp.float32),
        ],
        scratch_shapes=[pltpu.VMEM((1, D_MODEL), jnp.float32)],
        interpret=interpret,
    )(x, router_w, rb2, g2, lb2)

    grid_spec = pltpu.PrefetchScalarGridSpec(
        num_scalar_prefetch=1,
        grid=(B, F_CHUNKS),
        in_specs=[
            pl.BlockSpec((1, C, D_MODEL), lambda b, f, e: (b, 0, 0)),
            pl.BlockSpec((1, D_MODEL, F_BLK),
                         lambda b, f, e: (e[b, 0, 0], 0, f)),
            pl.BlockSpec((1, 1, F_BLK), lambda b, f, e: (e[b, 0, 0], 0, f)),
            pl.BlockSpec((1, F_BLK, D_MODEL),
                         lambda b, f, e: (e[b, 0, 0], f, 0)),
            pl.BlockSpec((1, 1, D_MODEL), lambda b, f, e: (e[b, 0, 0], 0, 0)),
            pl.BlockSpec((1, D_MODEL), lambda b, f, e: (0, 0)),
            pl.BlockSpec((1, D_MODEL), lambda b, f, e: (0, 0)),
            pl.BlockSpec((1, C, D_MODEL), lambda b, f, e: (b, 0, 0)),
        ],
        out_specs=pl.BlockSpec((1, C, D_MODEL), lambda b, f, e: (b, 0, 0)),
        scratch_shapes=[pltpu.VMEM((C, D_MODEL), jnp.float32)],
    )
    out = pl.pallas_call(
        _expert_kernel,
        grid_spec=grid_spec,
        out_shape=jax.ShapeDtypeStruct((B, S, D_MODEL), jnp.float32),
        input_output_aliases={8: 0},
        interpret=interpret,
    )(e_out, x, W1, b1r, W2, b2r, g2, lb2, out_a)
    return out


def kernel(x, router_w, router_b, W1, b1, W2, b2, ln_g, ln_b):
    return _run(x, router_w, router_b, W1, b1, W2, b2, ln_g, ln_b)


# speedup vs baseline: 1.1876x; 1.1876x over previous
import functools
import math

import jax
import jax.numpy as jnp
from jax.experimental import pallas as pl
from jax.experimental.pallas import tpu as pltpu

B = 2
S = 2048
D_MODEL = 768
D_FF = 2048
E = 64
CAP_FACTOR = 1.25
C = int(math.ceil(B * S * CAP_FACTOR / E))

S_CHUNKS = 8
S_BLK = S // S_CHUNKS
F_CHUNKS = 8
F_BLK = D_FF // F_CHUNKS


def _router_ln_kernel(x_ref, rw_ref, rb_ref, g_ref, bb_ref, e_ref, o_ref,
                      acc_ref):
    s = pl.program_id(1)
    xx = x_ref[0]
    part = jnp.sum(xx, axis=0, keepdims=True)

    @pl.when(s == 0)
    def _():
        acc_ref[...] = part

    @pl.when(s != 0)
    def _():
        acc_ref[...] += part

    mu = jnp.mean(xx, axis=1, keepdims=True)
    var = jnp.mean((xx - mu) ** 2, axis=1, keepdims=True)
    o_ref[0] = (xx - mu) * jax.lax.rsqrt(var + 1e-5) * g_ref[...] + bb_ref[...]

    @pl.when(s == S_CHUNKS - 1)
    def _():
        pooled = acc_ref[...] * (1.0 / S)
        logits = jnp.dot(pooled, rw_ref[...],
                         preferred_element_type=jnp.float32) + rb_ref[...]
        maxv = jnp.max(logits, axis=1, keepdims=True)
        idx = jax.lax.broadcasted_iota(jnp.int32, (1, E), 1)
        masked = jnp.where(logits >= maxv, idx, jnp.int32(E))
        am = jnp.min(masked, axis=1, keepdims=True)
        e_ref[0] = jnp.broadcast_to(am, (1, 128))


def _expert_kernel(e_ref, x_ref, w1_ref, b1_ref, w2_ref, b2_ref, g_ref,
                   bb_ref, prev_ref, o_ref):
    del prev_ref
    b = pl.program_id(0)
    xc = x_ref[0]
    h = jnp.maximum(
        jnp.dot(xc, w1_ref[0], preferred_element_type=jnp.float32)
        + b1_ref[0], 0.0)
    y = (jnp.dot(h, w2_ref[0], preferred_element_type=jnp.float32)
         + b2_ref[0])
    valid = jnp.logical_or(b == 0, e_ref[0, 0, 0] != e_ref[1, 0, 0])
    res = xc + jnp.where(valid, y, 0.0)
    mu = jnp.mean(res, axis=1, keepdims=True)
    var = jnp.mean((res - mu) ** 2, axis=1, keepdims=True)
    o_ref[0] = (res - mu) * jax.lax.rsqrt(var + 1e-5) * g_ref[...] + bb_ref[...]


@functools.partial(jax.jit, static_argnames=("interpret",))
def _run(x, router_w, router_b, W1, b1, W2, b2, ln_g, ln_b, interpret=False):
    rb2 = router_b.reshape(1, E)
    g2 = ln_g.reshape(1, D_MODEL)
    lb2 = ln_b.reshape(1, D_MODEL)
    b1r = b1.reshape(E, 1, D_FF)
    b2r = b2.reshape(E, 1, D_MODEL)

    e_out, out_a = pl.pallas_call(
        _router_ln_kernel,
        grid=(B, S_CHUNKS),
        in_specs=[
            pl.BlockSpec((1, S_BLK, D_MODEL), lambda b, s: (b, s, 0)),
            pl.BlockSpec((D_MODEL, E), lambda b, s: (0, 0)),
            pl.BlockSpec((1, E), lambda b, s: (0, 0)),
            pl.BlockSpec((1, D_MODEL), lambda b, s: (0, 0)),
            pl.BlockSpec((1, D_MODEL), lambda b, s: (0, 0)),
        ],
        out_specs=[
            pl.BlockSpec((1, 1, 128), lambda b, s: (b, 0, 0)),
            pl.BlockSpec((1, S_BLK, D_MODEL), lambda b, s: (b, s, 0)),
        ],
        out_shape=[
            jax.ShapeDtypeStruct((B, 1, 128), jnp.int32),
            jax.ShapeDtypeStruct((B, S, D_MODEL), jnp.float32),
        ],
        scratch_shapes=[pltpu.VMEM((1, D_MODEL), jnp.float32)],
        interpret=interpret,
    )(x, router_w, rb2, g2, lb2)

    grid_spec = pltpu.PrefetchScalarGridSpec(
        num_scalar_prefetch=1,
        grid=(B,),
        in_specs=[
            pl.BlockSpec((1, C, D_MODEL), lambda b, e: (b, 0, 0)),
            pl.BlockSpec((1, D_MODEL, D_FF), lambda b, e: (e[b, 0, 0], 0, 0)),
            pl.BlockSpec((1, 1, D_FF), lambda b, e: (e[b, 0, 0], 0, 0)),
            pl.BlockSpec((1, D_FF, D_MODEL), lambda b, e: (e[b, 0, 0], 0, 0)),
            pl.BlockSpec((1, 1, D_MODEL), lambda b, e: (e[b, 0, 0], 0, 0)),
            pl.BlockSpec((1, D_MODEL), lambda b, e: (0, 0)),
            pl.BlockSpec((1, D_MODEL), lambda b, e: (0, 0)),
            pl.BlockSpec((1, C, D_MODEL), lambda b, e: (b, 0, 0)),
        ],
        out_specs=pl.BlockSpec((1, C, D_MODEL), lambda b, e: (b, 0, 0)),
    )
    out = pl.pallas_call(
        _expert_kernel,
        grid_spec=grid_spec,
        out_shape=jax.ShapeDtypeStruct((B, S, D_MODEL), jnp.float32),
        input_output_aliases={8: 0},
        interpret=interpret,
    )(e_out, x, W1, b1r, W2, b2r, g2, lb2, out_a)
    return out


def kernel(x, router_w, router_b, W1, b1, W2, b2, ln_g, ln_b):
    return _run(x, router_w, router_b, W1, b1, W2, b2, ln_g, ln_b)


# speedup vs baseline: 1.4619x; 1.2310x over previous
import functools
import math

import jax
import jax.numpy as jnp
from jax.experimental import pallas as pl
from jax.experimental.pallas import tpu as pltpu

B = 2
S = 2048
D_MODEL = 768
D_FF = 2048
E = 64
CAP_FACTOR = 1.25
C = int(math.ceil(B * S * CAP_FACTOR / E))


def _router_ln_kernel(x_ref, rw_ref, rb_ref, g_ref, bb_ref, e_ref, o_ref):
    xx = x_ref[0]
    pooled = jnp.mean(xx, axis=0, keepdims=True)
    logits = jnp.dot(pooled, rw_ref[...],
                     preferred_element_type=jnp.float32) + rb_ref[...]
    maxv = jnp.max(logits, axis=1, keepdims=True)
    idx = jax.lax.broadcasted_iota(jnp.int32, (1, E), 1)
    masked = jnp.where(logits >= maxv, idx, jnp.int32(E))
    am = jnp.min(masked, axis=1, keepdims=True)
    e_ref[0] = jnp.broadcast_to(am, (8, 128))
    mu = jnp.mean(xx, axis=1, keepdims=True)
    var = jnp.mean((xx - mu) ** 2, axis=1, keepdims=True)
    o_ref[0] = (xx - mu) * jax.lax.rsqrt(var + 1e-5) * g_ref[...] + bb_ref[...]


def _expert_kernel(e_ref, x_ref, w1_ref, b1_ref, w2_ref, b2_ref, g_ref,
                   bb_ref, prev_ref, o_ref):
    del prev_ref
    b = pl.program_id(0)
    xc = x_ref[0]
    xb = xc.astype(jnp.bfloat16)
    h = jnp.maximum(
        jnp.dot(xb, w1_ref[0].astype(jnp.bfloat16),
                preferred_element_type=jnp.float32)
        + b1_ref[0], 0.0)
    y = (jnp.dot(h.astype(jnp.bfloat16), w2_ref[0].astype(jnp.bfloat16),
                 preferred_element_type=jnp.float32)
         + b2_ref[0])
    valid = jnp.logical_or(b == 0, e_ref[0] != e_ref[1])
    res = xc + jnp.where(valid, y, 0.0)
    mu = jnp.mean(res, axis=1, keepdims=True)
    var = jnp.mean((res - mu) ** 2, axis=1, keepdims=True)
    o_ref[0] = (res - mu) * jax.lax.rsqrt(var + 1e-5) * g_ref[...] + bb_ref[...]


@functools.partial(jax.jit, static_argnames=("interpret",))
def _run(x, router_w, router_b, W1, b1, W2, b2, ln_g, ln_b, interpret=False):
    rb2 = router_b.reshape(1, E)
    g2 = ln_g.reshape(1, D_MODEL)
    lb2 = ln_b.reshape(1, D_MODEL)
    b1r = b1.reshape(E, 1, D_FF)
    b2r = b2.reshape(E, 1, D_MODEL)

    e_out, out_a = pl.pallas_call(
        _router_ln_kernel,
        grid=(B,),
        in_specs=[
            pl.BlockSpec((1, S, D_MODEL), lambda b: (b, 0, 0)),
            pl.BlockSpec((D_MODEL, E), lambda b: (0, 0)),
            pl.BlockSpec((1, E), lambda b: (0, 0)),
            pl.BlockSpec((1, D_MODEL), lambda b: (0, 0)),
            pl.BlockSpec((1, D_MODEL), lambda b: (0, 0)),
        ],
        out_specs=[
            pl.BlockSpec((1, 8, 128), lambda b: (b, 0, 0)),
            pl.BlockSpec((1, S, D_MODEL), lambda b: (b, 0, 0)),
        ],
        out_shape=[
            jax.ShapeDtypeStruct((B, 8, 128), jnp.int32),
            jax.ShapeDtypeStruct((B, S, D_MODEL), jnp.float32),
        ],
        interpret=interpret,
    )(x, router_w, rb2, g2, lb2)
    e_idx = e_out[:, 0, 0]

    grid_spec = pltpu.PrefetchScalarGridSpec(
        num_scalar_prefetch=1,
        grid=(B,),
        in_specs=[
            pl.BlockSpec((1, C, D_MODEL), lambda b, e: (b, 0, 0)),
            pl.BlockSpec((1, D_MODEL, D_FF), lambda b, e: (e[b], 0, 0)),
            pl.BlockSpec((1, 1, D_FF), lambda b, e: (e[b], 0, 0)),
            pl.BlockSpec((1, D_FF, D_MODEL), lambda b, e: (e[b], 0, 0)),
            pl.BlockSpec((1, 1, D_MODEL), lambda b, e: (e[b], 0, 0)),
            pl.BlockSpec((1, D_MODEL), lambda b, e: (0, 0)),
            pl.BlockSpec((1, D_MODEL), lambda b, e: (0, 0)),
            pl.BlockSpec((1, C, D_MODEL), lambda b, e: (b, 0, 0)),
        ],
        out_specs=pl.BlockSpec((1, C, D_MODEL), lambda b, e: (b, 0, 0)),
    )
    out = pl.pallas_call(
        _expert_kernel,
        grid_spec=grid_spec,
        out_shape=jax.ShapeDtypeStruct((B, S, D_MODEL), jnp.float32),
        input_output_aliases={8: 0},
        interpret=interpret,
    )(e_idx, x, W1, b1r, W2, b2r, g2, lb2, out_a)
    return out


def kernel(x, router_w, router_b, W1, b1, W2, b2, ln_g, ln_b):
    return _run(x, router_w, router_b, W1, b1, W2, b2, ln_g, ln_b)


# speedup vs baseline: 1.4795x; 1.0120x over previous
import functools
import math

import jax
import jax.numpy as jnp
from jax.experimental import pallas as pl
from jax.experimental.pallas import tpu as pltpu

B = 2
S = 2048
D_MODEL = 768
D_FF = 2048
E = 64
CAP_FACTOR = 1.25
C = int(math.ceil(B * S * CAP_FACTOR / E))


def _fused_kernel(x_ref, rw_ref, rb_ref, g_ref, bb_ref,
                  w1_hbm, b1_hbm, w2_hbm, b2_hbm,
                  o_ref,
                  e_smem, w1_v, b1_v, w2_v, b2_v, sems):
    b = pl.program_id(0)
    xx = x_ref[0]

    pooled = jnp.mean(xx, axis=0, keepdims=True)
    logits = jnp.dot(pooled, rw_ref[...],
                     preferred_element_type=jnp.float32) + rb_ref[...]
    maxv = jnp.max(logits)
    idx = jax.lax.broadcasted_iota(jnp.int32, (1, E), 1)
    masked = jnp.where(logits >= maxv, idx, jnp.int32(E))
    am = jnp.min(masked)
    e_smem[b] = am

    cp1 = pltpu.make_async_copy(w1_hbm.at[pl.ds(am, 1)], w1_v, sems.at[0])
    cp2 = pltpu.make_async_copy(w2_hbm.at[pl.ds(am, 1)], w2_v, sems.at[1])
    cp3 = pltpu.make_async_copy(b1_hbm.at[pl.ds(am, 1)], b1_v, sems.at[2])
    cp4 = pltpu.make_async_copy(b2_hbm.at[pl.ds(am, 1)], b2_v, sems.at[3])
    cp1.start()
    cp2.start()
    cp3.start()
    cp4.start()

    g = g_ref[...]
    bb = bb_ref[...]

    rest = xx[C:]
    mu_r = jnp.mean(rest, axis=1, keepdims=True)
    var_r = jnp.mean((rest - mu_r) ** 2, axis=1, keepdims=True)
    o_ref[0, C:, :] = (rest - mu_r) * jax.lax.rsqrt(var_r + 1e-5) * g + bb

    cp1.wait()
    cp2.wait()
    cp3.wait()
    cp4.wait()

    xc = xx[:C]
    h = jnp.maximum(
        jnp.dot(xc.astype(jnp.bfloat16), w1_v[0].astype(jnp.bfloat16),
                preferred_element_type=jnp.float32) + b1_v[0], 0.0)
    y = (jnp.dot(h.astype(jnp.bfloat16), w2_v[0].astype(jnp.bfloat16),
                 preferred_element_type=jnp.float32) + b2_v[0])
    valid = jnp.logical_or(b == 0, e_smem[0] != e_smem[1])
    res = xc + jnp.where(valid, y, 0.0)
    mu = jnp.mean(res, axis=1, keepdims=True)
    var = jnp.mean((res - mu) ** 2, axis=1, keepdims=True)
    o_ref[0, :C, :] = (res - mu) * jax.lax.rsqrt(var + 1e-5) * g + bb


@functools.partial(jax.jit, static_argnames=("interpret",))
def _run(x, router_w, router_b, W1, b1, W2, b2, ln_g, ln_b, interpret=False):
    rb2 = router_b.reshape(1, E)
    g2 = ln_g.reshape(1, D_MODEL)
    lb2 = ln_b.reshape(1, D_MODEL)
    b1r = b1.reshape(E, 1, D_FF)
    b2r = b2.reshape(E, 1, D_MODEL)

    out = pl.pallas_call(
        _fused_kernel,
        grid=(B,),
        in_specs=[
            pl.BlockSpec((1, S, D_MODEL), lambda b: (b, 0, 0)),
            pl.BlockSpec((D_MODEL, E), lambda b: (0, 0)),
            pl.BlockSpec((1, E), lambda b: (0, 0)),
            pl.BlockSpec((1, D_MODEL), lambda b: (0, 0)),
            pl.BlockSpec((1, D_MODEL), lambda b: (0, 0)),
            pl.BlockSpec(memory_space=pltpu.MemorySpace.HBM),
            pl.BlockSpec(memory_space=pltpu.MemorySpace.HBM),
            pl.BlockSpec(memory_space=pltpu.MemorySpace.HBM),
            pl.BlockSpec(memory_space=pltpu.MemorySpace.HBM),
        ],
        out_specs=pl.BlockSpec((1, S, D_MODEL), lambda b: (b, 0, 0)),
        out_shape=jax.ShapeDtypeStruct((B, S, D_MODEL), jnp.float32),
        scratch_shapes=[
            pltpu.SMEM((B,), jnp.int32),
            pltpu.VMEM((1, D_MODEL, D_FF), jnp.float32),
            pltpu.VMEM((1, 1, D_FF), jnp.float32),
            pltpu.VMEM((1, D_FF, D_MODEL), jnp.float32),
            pltpu.VMEM((1, 1, D_MODEL), jnp.float32),
            pltpu.SemaphoreType.DMA((4,)),
        ],
        interpret=interpret,
    )(x, router_w, rb2, g2, lb2, W1, b1r, W2, b2r)
    return out


def kernel(x, router_w, router_b, W1, b1, W2, b2, ln_g, ln_b):
    return _run(x, router_w, router_b, W1, b1, W2, b2, ln_g, ln_b)


# speedup vs baseline: 1.4983x; 1.0127x over previous
import functools
import math

import jax
import jax.numpy as jnp
from jax.experimental import pallas as pl
from jax.experimental.pallas import tpu as pltpu

B = 2
S = 2048
D_MODEL = 768
D_FF = 2048
E = 64
CAP_FACTOR = 1.25
C = int(math.ceil(B * S * CAP_FACTOR / E))

W1_CH = 4
W2_CH = 4
LNEPS = 1e-5


def _ln(v, g, bb):
    mu = jnp.mean(v, axis=1, keepdims=True)
    m2 = jnp.mean(v * v, axis=1, keepdims=True)
    k = jax.lax.rsqrt(m2 - mu * mu + LNEPS)
    return (v - mu) * k * g + bb


def _fused_kernel(x_ref, rw_ref, rb_ref, g_ref, bb_ref,
                  w1_hbm, b1_hbm, w2_hbm, b2_hbm,
                  o_ref,
                  e_smem, w1_v, b1_v, w2_v, b2_v, sem1, sem2, semb):
    b = pl.program_id(0)
    xx = x_ref[0]

    pooled = jnp.mean(xx, axis=0, keepdims=True)
    logits = jnp.dot(pooled, rw_ref[...],
                     preferred_element_type=jnp.float32) + rb_ref[...]
    maxv = jnp.max(logits)
    idx = jax.lax.broadcasted_iota(jnp.int32, (1, E), 1)
    masked = jnp.where(logits >= maxv, idx, jnp.int32(E))
    am = jnp.min(masked)
    e_smem[b] = am

    r1 = D_MODEL // W1_CH
    cps1 = [pltpu.make_async_copy(
        w1_hbm.at[pl.ds(am, 1), pl.ds(k * r1, r1), :],
        w1_v.at[:, pl.ds(k * r1, r1), :], sem1.at[k]) for k in range(W1_CH)]
    r2 = D_FF // W2_CH
    cps2 = [pltpu.make_async_copy(
        w2_hbm.at[pl.ds(am, 1), pl.ds(k * r2, r2), :],
        w2_v.at[:, pl.ds(k * r2, r2), :], sem2.at[k]) for k in range(W2_CH)]
    cpb1 = pltpu.make_async_copy(b1_hbm.at[pl.ds(am, 1)], b1_v, semb.at[0])
    cpb2 = pltpu.make_async_copy(b2_hbm.at[pl.ds(am, 1)], b2_v, semb.at[1])
    for cp in cps1:
        cp.start()
    for cp in cps2:
        cp.start()
    cpb1.start()
    cpb2.start()

    g = g_ref[...]
    bb = bb_ref[...]

    o_ref[0, C:, :] = _ln(xx[C:], g, bb)

    for cp in cps1:
        cp.wait()
    cpb1.wait()
    xc = xx[:C]
    h = jnp.maximum(
        jnp.dot(xc.astype(jnp.bfloat16), w1_v[0].astype(jnp.bfloat16),
                preferred_element_type=jnp.float32) + b1_v[0], 0.0)
    for cp in cps2:
        cp.wait()
    cpb2.wait()
    y = (jnp.dot(h.astype(jnp.bfloat16), w2_v[0].astype(jnp.bfloat16),
                 preferred_element_type=jnp.float32) + b2_v[0])
    valid = jnp.logical_or(b == 0, e_smem[0] != e_smem[1])
    res = xc + jnp.where(valid, y, 0.0)
    o_ref[0, :C, :] = _ln(res, g, bb)


@functools.partial(jax.jit, static_argnames=("interpret",))
def _run(x, router_w, router_b, W1, b1, W2, b2, ln_g, ln_b, interpret=False):
    rb2 = router_b.reshape(1, E)
    g2 = ln_g.reshape(1, D_MODEL)
    lb2 = ln_b.reshape(1, D_MODEL)
    b1r = b1.reshape(E, 1, D_FF)
    b2r = b2.reshape(E, 1, D_MODEL)

    out = pl.pallas_call(
        _fused_kernel,
        grid=(B,),
        in_specs=[
            pl.BlockSpec((1, S, D_MODEL), lambda b: (b, 0, 0)),
            pl.BlockSpec((D_MODEL, E), lambda b: (0, 0)),
            pl.BlockSpec((1, E), lambda b: (0, 0)),
            pl.BlockSpec((1, D_MODEL), lambda b: (0, 0)),
            pl.BlockSpec((1, D_MODEL), lambda b: (0, 0)),
            pl.BlockSpec(memory_space=pltpu.MemorySpace.HBM),
            pl.BlockSpec(memory_space=pltpu.MemorySpace.HBM),
            pl.BlockSpec(memory_space=pltpu.MemorySpace.HBM),
            pl.BlockSpec(memory_space=pltpu.MemorySpace.HBM),
        ],
        out_specs=pl.BlockSpec((1, S, D_MODEL), lambda b: (b, 0, 0)),
        out_shape=jax.ShapeDtypeStruct((B, S, D_MODEL), jnp.float32),
        scratch_shapes=[
            pltpu.SMEM((B,), jnp.int32),
            pltpu.VMEM((1, D_MODEL, D_FF), jnp.float32),
            pltpu.VMEM((1, 1, D_FF), jnp.float32),
            pltpu.VMEM((1, D_FF, D_MODEL), jnp.float32),
            pltpu.VMEM((1, 1, D_MODEL), jnp.float32),
            pltpu.SemaphoreType.DMA((W1_CH,)),
            pltpu.SemaphoreType.DMA((W2_CH,)),
            pltpu.SemaphoreType.DMA((2,)),
        ],
        interpret=interpret,
    )(x, router_w, rb2, g2, lb2, W1, b1r, W2, b2r)
    return out


def kernel(x, router_w, router_b, W1, b1, W2, b2, ln_g, ln_b):
    return _run(x, router_w, router_b, W1, b1, W2, b2, ln_g, ln_b)


# speedup vs baseline: 1.5914x; 1.0622x over previous
import functools
import math

import jax
import jax.numpy as jnp
from jax.experimental import pallas as pl
from jax.experimental.pallas import tpu as pltpu

B = 2
S = 2048
D_MODEL = 768
D_FF = 2048
E = 64
CAP_FACTOR = 1.25
C = int(math.ceil(B * S * CAP_FACTOR / E))

W1_CH = 4
W2_CH = 4
LNEPS = 1e-5


def _ln(v, g, bb):
    mu = jnp.mean(v, axis=1, keepdims=True)
    m2 = jnp.mean(v * v, axis=1, keepdims=True)
    k = jax.lax.rsqrt(m2 - mu * mu + LNEPS)
    return (v - mu) * k * g + bb


def _router(x_v, b, rw, rb):
    pooled = jnp.mean(x_v[b], axis=0, keepdims=True)
    logits = jnp.dot(pooled, rw,
                     preferred_element_type=jnp.float32) + rb
    maxv = jnp.max(logits)
    idx = jax.lax.broadcasted_iota(jnp.int32, (1, E), 1)
    masked = jnp.where(logits >= maxv, idx, jnp.int32(E))
    return jnp.min(masked)


def _fused_kernel(x_hbm, rw_ref, rb_ref, g_ref, bb_ref,
                  w1_hbm, b1_hbm, w2_hbm, b2_hbm,
                  o_hbm,
                  x_v, w1_v, b1_v, w2_v, b2_v, head_v,
                  semx, sem1, sem2, semb, semo, semh):
    cpx = [pltpu.make_async_copy(x_hbm.at[pl.ds(b, 1)], x_v.at[pl.ds(b, 1)],
                                 semx.at[b]) for b in range(B)]
    cpx[0].start()
    cpx[1].start()

    rw = rw_ref[...]
    rb = rb_ref[...]
    g = g_ref[...]
    bb = bb_ref[...]

    r1 = D_MODEL // W1_CH
    r2 = D_FF // W2_CH

    ams = []
    wcps = []
    for b in range(B):
        cpx[b].wait()
        am = _router(x_v, b, rw, rb)
        ams.append(am)
        cps1 = [pltpu.make_async_copy(
            w1_hbm.at[pl.ds(am, 1), pl.ds(k * r1, r1), :],
            w1_v.at[pl.ds(b, 1), pl.ds(k * r1, r1), :],
            sem1.at[b, k]) for k in range(W1_CH)]
        cps2 = [pltpu.make_async_copy(
            w2_hbm.at[pl.ds(am, 1), pl.ds(k * r2, r2), :],
            w2_v.at[pl.ds(b, 1), pl.ds(k * r2, r2), :],
            sem2.at[b, k]) for k in range(W2_CH)]
        cpb1 = pltpu.make_async_copy(b1_hbm.at[pl.ds(am, 1)],
                                     b1_v.at[pl.ds(b, 1)], semb.at[b, 0])
        cpb2 = pltpu.make_async_copy(b2_hbm.at[pl.ds(am, 1)],
                                     b2_v.at[pl.ds(b, 1)], semb.at[b, 1])
        for cp in cps1 + cps2 + [cpb1, cpb2]:
            cp.start()
        wcps.append(cps1 + cps2 + [cpb1, cpb2])

    cpo = []
    for b in range(B):
        x_v[b, C:, :] = _ln(x_v[b, C:, :], g, bb)
        cp = pltpu.make_async_copy(
            x_v.at[pl.ds(b, 1), pl.ds(C, S - C), :],
            o_hbm.at[pl.ds(b, 1), pl.ds(C, S - C), :], semo.at[b])
        cp.start()
        cpo.append(cp)

    cph = []
    for b in range(B):
        for cp in wcps[b]:
            cp.wait()
        xc = x_v[b, :C, :]
        h = jnp.maximum(
            jnp.dot(xc.astype(jnp.bfloat16), w1_v[b].astype(jnp.bfloat16),
                    preferred_element_type=jnp.float32) + b1_v[b], 0.0)
        y = (jnp.dot(h.astype(jnp.bfloat16), w2_v[b].astype(jnp.bfloat16),
                     preferred_element_type=jnp.float32) + b2_v[b])
        if b == 1:
            y = jnp.where(ams[0] != ams[1], y, 0.0)
        head_v[b] = _ln(xc + y, g, bb)
        cp = pltpu.make_async_copy(
            head_v.at[pl.ds(b, 1)],
            o_hbm.at[pl.ds(b, 1), pl.ds(0, C), :], semh.at[b])
        cp.start()
        cph.append(cp)

    for cp in cpo + cph:
        cp.wait()


@functools.partial(jax.jit, static_argnames=("interpret",))
def _run(x, router_w, router_b, W1, b1, W2, b2, ln_g, ln_b, interpret=False):
    rb2 = router_b.reshape(1, E)
    g2 = ln_g.reshape(1, D_MODEL)
    lb2 = ln_b.reshape(1, D_MODEL)
    b1r = b1.reshape(E, 1, D_FF)
    b2r = b2.reshape(E, 1, D_MODEL)

    out = pl.pallas_call(
        _fused_kernel,
        in_specs=[
            pl.BlockSpec(memory_space=pltpu.MemorySpace.HBM),
            pl.BlockSpec(memory_space=pltpu.MemorySpace.VMEM),
            pl.BlockSpec(memory_space=pltpu.MemorySpace.VMEM),
            pl.BlockSpec(memory_space=pltpu.MemorySpace.VMEM),
            pl.BlockSpec(memory_space=pltpu.MemorySpace.VMEM),
            pl.BlockSpec(memory_space=pltpu.MemorySpace.HBM),
            pl.BlockSpec(memory_space=pltpu.MemorySpace.HBM),
            pl.BlockSpec(memory_space=pltpu.MemorySpace.HBM),
            pl.BlockSpec(memory_space=pltpu.MemorySpace.HBM),
        ],
        out_specs=pl.BlockSpec(memory_space=pltpu.MemorySpace.HBM),
        out_shape=jax.ShapeDtypeStruct((B, S, D_MODEL), jnp.float32),
        scratch_shapes=[
            pltpu.VMEM((B, S, D_MODEL), jnp.float32),
            pltpu.VMEM((B, D_MODEL, D_FF), jnp.float32),
            pltpu.VMEM((B, 1, D_FF), jnp.float32),
            pltpu.VMEM((B, D_FF, D_MODEL), jnp.float32),
            pltpu.VMEM((B, 1, D_MODEL), jnp.float32),
            pltpu.VMEM((B, C, D_MODEL), jnp.float32),
            pltpu.SemaphoreType.DMA((B,)),
            pltpu.SemaphoreType.DMA((B, W1_CH)),
            pltpu.SemaphoreType.DMA((B, W2_CH)),
            pltpu.SemaphoreType.DMA((B, 2)),
            pltpu.SemaphoreType.DMA((B,)),
            pltpu.SemaphoreType.DMA((B,)),
        ],
        interpret=interpret,
    )(x, router_w, rb2, g2, lb2, W1, b1r, W2, b2r)
    return out


def kernel(x, router_w, router_b, W1, b1, W2, b2, ln_g, ln_b):
    return _run(x, router_w, router_b, W1, b1, W2, b2, ln_g, ln_b)
